# Initial kernel scaffold; baseline (speedup 1.0000x reference)
import jax
import jax.numpy as jnp
from jax.experimental import pallas as pl


def kernel(x, edge_index, W1, b1, W2, b2):
    raise NotImplementedError("write your pallas kernel here")


# speedup vs baseline: 6.8009x; 6.8009x over previous
import jax
import jax.numpy as jnp
from jax import lax
from jax.experimental import pallas as pl
from jax.experimental.pallas import tpu as pltpu
from jax.experimental.pallas import tpu_sc as plsc

NC = 2
NS = 16
NW = NC * NS
LANES = 16
CHUNK = 80
JB = 25


def _make_sc_segsum(n, e, d, with_deg):
    n_chunks = e // CHUNK
    chunks_per_tile = n_chunks // NW
    outer = chunks_per_tile // JB
    rows_per_tile = n // NS

    mesh = plsc.VectorSubcoreMesh(core_axis_name="c", subcore_axis_name="s")

    out_type = [jax.ShapeDtypeStruct((NC, NS, rows_per_tile, d), jnp.float32)]
    scratch = [
        pltpu.VMEM((JB, CHUNK), jnp.int32),
        pltpu.VMEM((JB, CHUNK), jnp.int32),
        pltpu.VMEM((CHUNK, d), jnp.float32),
        pltpu.VMEM_SHARED((n, d), jnp.float32),
        pltpu.SemaphoreType.DMA,
    ]
    if with_deg:
        out_type.append(jax.ShapeDtypeStruct((NW, n), jnp.float32))
        scratch.append(pltpu.VMEM((n,), jnp.float32))

    def body(x_hbm, rows_hbm, cols_hbm, zrows_hbm, zn_hbm, agg_out, deg_out,
             rowbuf, colbuf, databuf, aggs, sem, degbuf):
        cid = lax.axis_index("c")
        sid = lax.axis_index("s")
        wid = cid * NS + sid
        r0 = sid * rows_per_tile
        pltpu.sync_copy(zrows_hbm, aggs.at[pl.ds(r0, rows_per_tile)])
        if with_deg:
            pltpu.sync_copy(zn_hbm, degbuf)
        plsc.subcore_barrier()

        ones = jnp.full((LANES,), 1.0, jnp.float32)

        def outer_body(ob, carry):
            pltpu.sync_copy(rows_hbm.at[wid, ob], rowbuf)
            pltpu.sync_copy(cols_hbm.at[wid, ob], colbuf)
            if with_deg:
                def degloop(j, carry2):
                    for i in range(CHUNK // LANES):
                        rv = rowbuf[j, pl.ds(i * LANES, LANES)]
                        plsc.addupdate_scatter(degbuf, [rv], ones)
                    return carry2
                lax.fori_loop(0, JB, degloop, 0)

            def inner(j, carry2):
                pltpu.async_copy(x_hbm.at[colbuf.at[j]], databuf, sem).wait()
                pltpu.sync_copy(databuf, aggs.at[rowbuf.at[j]], add=True)
                return carry2
            lax.fori_loop(0, JB, inner, 0)
            return carry
        lax.fori_loop(0, outer, outer_body, 0)

        if with_deg:
            pltpu.sync_copy(degbuf, deg_out.at[wid])
        plsc.subcore_barrier()
        pltpu.sync_copy(aggs.at[pl.ds(r0, rows_per_tile)],
                        agg_out.at[cid, sid])

    if with_deg:
        def full_body(x_hbm, rows_hbm, cols_hbm, zrows_hbm, zn_hbm,
                      agg_out, deg_out, rowbuf, colbuf, databuf, aggs, sem,
                      degbuf):
            body(x_hbm, rows_hbm, cols_hbm, zrows_hbm, zn_hbm, agg_out,
                 deg_out, rowbuf, colbuf, databuf, aggs, sem, degbuf)
    else:
        def full_body(x_hbm, rows_hbm, cols_hbm, zrows_hbm, agg_out,
                      rowbuf, colbuf, databuf, aggs, sem):
            body(x_hbm, rows_hbm, cols_hbm, zrows_hbm, None, agg_out,
                 None, rowbuf, colbuf, databuf, aggs, sem, None)

    return pl.kernel(full_body, out_type=out_type, mesh=mesh,
                     scratch_types=scratch,
                     compiler_params=pltpu.CompilerParams(
                         needs_layout_passes=False))


def _tc_layer(aggp, degp, x, w, b, relu):
    n, d = x.shape
    dout = w.shape[1]
    bn = 1000

    def body(aggp_ref, degp_ref, x_ref, w_ref, b_ref, o_ref):
        agg = aggp_ref[0] + aggp_ref[1]
        deg = jnp.maximum(jnp.sum(degp_ref[...], axis=1), 1.0)
        z = agg / deg[:, None] + x_ref[...]
        z = jnp.dot(z, w_ref[...], preferred_element_type=jnp.float32)
        z = z + b_ref[...]
        if relu:
            z = jnp.maximum(z, 0.0)
        o_ref[...] = z

    return pl.pallas_call(
        body,
        grid=(n // bn,),
        in_specs=[
            pl.BlockSpec((NC, bn, d), lambda i: (0, i, 0)),
            pl.BlockSpec((bn, NW), lambda i: (i, 0)),
            pl.BlockSpec((bn, d), lambda i: (i, 0)),
            pl.BlockSpec((d, dout), lambda i: (0, 0)),
            pl.BlockSpec((1, dout), lambda i: (0, 0)),
        ],
        out_specs=pl.BlockSpec((bn, dout), lambda i: (i, 0)),
        out_shape=jax.ShapeDtypeStruct((n, dout), jnp.float32),
    )(aggp, degp, x, w, b)


def kernel(x, edge_index, W1, b1, W2, b2):
    n, d = x.shape
    e = edge_index.shape[1]
    cpt = e // (NW * CHUNK)
    rows = edge_index[0].reshape(NW, cpt // JB, JB, CHUNK)
    cols = edge_index[1].reshape(NW, cpt // JB, JB, CHUNK)
    zrows = jnp.zeros((n // NS, d), jnp.float32)
    zn = jnp.zeros((n,), jnp.float32)

    agg1, degp = _make_sc_segsum(n, e, d, True)(x, rows, cols, zrows, zn)
    degp_t = degp.T
    h = _tc_layer(agg1.reshape(NC, n, d), degp_t, x, W1, b1.reshape(1, -1),
                  True)
    (agg2,) = _make_sc_segsum(n, e, d, False)(h, rows, cols, zrows)
    out = _tc_layer(agg2.reshape(NC, n, d), degp_t, h, W2, b2.reshape(1, -1),
                    False)
    return out


# speedup vs baseline: 11.1452x; 1.6388x over previous
import jax
import jax.numpy as jnp
from jax import lax
from jax.experimental import pallas as pl
from jax.experimental.pallas import tpu as pltpu
from jax.experimental.pallas import tpu_sc as plsc

NC = 2
NS = 16
NW = NC * NS
LANES = 16
CHUNK = 80
JB = 25


def _make_sc_segsum(n, e, d, with_deg):
    chunks_per_tile = e // (NW * CHUNK)
    outer = chunks_per_tile // JB
    rows_per_tile = n // NS

    mesh = plsc.VectorSubcoreMesh(core_axis_name="c", subcore_axis_name="s")

    out_type = [jax.ShapeDtypeStruct((NC, NS, rows_per_tile, d), jnp.float32)]
    scratch = [
        pltpu.VMEM((JB, CHUNK), jnp.int32),
        pltpu.VMEM((JB, CHUNK), jnp.int32),
        pltpu.VMEM((2, CHUNK, d), jnp.float32),
        pltpu.VMEM_SHARED((n, d), jnp.float32),
        pltpu.SemaphoreType.DMA((2,)),
    ]
    if with_deg:
        out_type.append(jax.ShapeDtypeStruct((NW, n), jnp.float32))
        scratch.append(pltpu.VMEM((n,), jnp.float32))

    def body(x_hbm, rows_hbm, cols_hbm, zrows_hbm, zn_hbm, agg_out, deg_out,
             rowbuf, colbuf, databuf, aggs, sem, degbuf):
        cid = lax.axis_index("c")
        sid = lax.axis_index("s")
        wid = cid * NS + sid
        r0 = sid * rows_per_tile
        pltpu.sync_copy(zrows_hbm, aggs.at[pl.ds(r0, rows_per_tile)])
        if with_deg:
            pltpu.sync_copy(zn_hbm, degbuf)
        plsc.subcore_barrier()

        ones = jnp.full((LANES,), 1.0, jnp.float32)

        def outer_body(ob, carry):
            pltpu.sync_copy(rows_hbm.at[wid, ob], rowbuf)
            pltpu.sync_copy(cols_hbm.at[wid, ob], colbuf)
            if with_deg:
                def degloop(j, carry2):
                    for i in range(CHUNK // LANES):
                        rv = rowbuf[j, pl.ds(i * LANES, LANES)]
                        plsc.addupdate_scatter(degbuf, [rv], ones)
                    return carry2
                lax.fori_loop(0, JB, degloop, 0)

            pltpu.async_copy(x_hbm.at[colbuf.at[0]], databuf.at[0],
                             sem.at[0])

            def inner(j, carry2):
                p = lax.rem(j, 2)
                q = lax.rem(j + 1, 2)

                @pl.when(j + 1 < JB)
                def _():
                    pltpu.async_copy(x_hbm.at[colbuf.at[j + 1]],
                                     databuf.at[q], sem.at[q])

                pltpu.make_async_copy(x_hbm.at[colbuf.at[j]],
                                      databuf.at[p], sem.at[p]).wait()
                pltpu.sync_copy(databuf.at[p], aggs.at[rowbuf.at[j]],
                                add=True)
                return carry2
            lax.fori_loop(0, JB, inner, 0)
            return carry
        lax.fori_loop(0, outer, outer_body, 0)

        if with_deg:
            pltpu.sync_copy(degbuf, deg_out.at[wid])
        plsc.subcore_barrier()
        pltpu.sync_copy(aggs.at[pl.ds(r0, rows_per_tile)],
                        agg_out.at[cid, sid])

    if with_deg:
        def full_body(x_hbm, rows_hbm, cols_hbm, zrows_hbm, zn_hbm,
                      agg_out, deg_out, rowbuf, colbuf, databuf, aggs, sem,
                      degbuf):
            body(x_hbm, rows_hbm, cols_hbm, zrows_hbm, zn_hbm, agg_out,
                 deg_out, rowbuf, colbuf, databuf, aggs, sem, degbuf)
    else:
        def full_body(x_hbm, rows_hbm, cols_hbm, zrows_hbm, agg_out,
                      rowbuf, colbuf, databuf, aggs, sem):
            body(x_hbm, rows_hbm, cols_hbm, zrows_hbm, None, agg_out,
                 None, rowbuf, colbuf, databuf, aggs, sem, None)

    return pl.kernel(full_body, out_type=out_type, mesh=mesh,
                     scratch_types=scratch,
                     compiler_params=pltpu.CompilerParams(
                         needs_layout_passes=False,
                         use_tc_tiling_on_sc=False))


_BN = 1000


def _tc_matmul(x, w):
    n, d = x.shape
    dout = w.shape[1]

    def body(x_ref, w_ref, o_ref):
        o_ref[...] = jnp.dot(x_ref[...], w_ref[...],
                             preferred_element_type=jnp.float32)

    return pl.pallas_call(
        body,
        grid=(n // _BN,),
        in_specs=[
            pl.BlockSpec((_BN, d), lambda i: (i, 0)),
            pl.BlockSpec((d, dout), lambda i: (0, 0)),
        ],
        out_specs=pl.BlockSpec((_BN, dout), lambda i: (i, 0)),
        out_shape=jax.ShapeDtypeStruct((n, dout), jnp.float32),
    )(x, w)


def _tc_mid(aggp, degp_t, y1, b1, w2):
    n, d = y1.shape
    dout = w2.shape[1]

    def body(aggp_ref, degp_ref, y_ref, b_ref, w_ref, o_ref):
        agg = aggp_ref[0] + aggp_ref[1]
        deg = jnp.maximum(jnp.sum(degp_ref[...], axis=1), 1.0)
        h = agg / deg[:, None] + y_ref[...] + b_ref[...]
        h = jnp.maximum(h, 0.0)
        o_ref[...] = jnp.dot(h, w_ref[...],
                             preferred_element_type=jnp.float32)

    return pl.pallas_call(
        body,
        grid=(n // _BN,),
        in_specs=[
            pl.BlockSpec((NC, _BN, d), lambda i: (0, i, 0)),
            pl.BlockSpec((_BN, NW), lambda i: (i, 0)),
            pl.BlockSpec((_BN, d), lambda i: (i, 0)),
            pl.BlockSpec((1, d), lambda i: (0, 0)),
            pl.BlockSpec((d, dout), lambda i: (0, 0)),
        ],
        out_specs=pl.BlockSpec((_BN, dout), lambda i: (i, 0)),
        out_shape=jax.ShapeDtypeStruct((n, dout), jnp.float32),
    )(aggp, degp_t, y1, b1, w2)


def _tc_final(aggp, degp_t, y2, b2):
    n, d = y2.shape

    def body(aggp_ref, degp_ref, y_ref, b_ref, o_ref):
        agg = aggp_ref[0] + aggp_ref[1]
        deg = jnp.maximum(jnp.sum(degp_ref[...], axis=1), 1.0)
        o_ref[...] = agg / deg[:, None] + y_ref[...] + b_ref[...]

    return pl.pallas_call(
        body,
        grid=(n // _BN,),
        in_specs=[
            pl.BlockSpec((NC, _BN, d), lambda i: (0, i, 0)),
            pl.BlockSpec((_BN, NW), lambda i: (i, 0)),
            pl.BlockSpec((_BN, d), lambda i: (i, 0)),
            pl.BlockSpec((1, d), lambda i: (0, 0)),
        ],
        out_specs=pl.BlockSpec((_BN, d), lambda i: (i, 0)),
        out_shape=jax.ShapeDtypeStruct((n, d), jnp.float32),
    )(aggp, degp_t, y2, b2)


def kernel(x, edge_index, W1, b1, W2, b2):
    n, d = x.shape
    e = edge_index.shape[1]
    d2 = W2.shape[1]
    cpt = e // (NW * CHUNK)
    rows = edge_index[0].reshape(NW, cpt // JB, JB, CHUNK)
    cols = edge_index[1].reshape(NW, cpt // JB, JB, CHUNK)
    zrows = jnp.zeros((n // NS, d), jnp.float32)
    zrows2 = jnp.zeros((n // NS, d2), jnp.float32)
    zn = jnp.zeros((n,), jnp.float32)

    y1 = _tc_matmul(x, W1)
    agg1, degp = _make_sc_segsum(n, e, d, True)(y1, rows, cols, zrows, zn)
    degp_t = degp.T
    y2 = _tc_mid(agg1.reshape(NC, n, d), degp_t, y1, b1.reshape(1, -1), W2)
    (agg2,) = _make_sc_segsum(n, e, d2, False)(y2, rows, cols, zrows2)
    out = _tc_final(agg2.reshape(NC, n, d2), degp_t, y2, b2.reshape(1, -1))
    return out


# speedup vs baseline: 11.5705x; 1.0382x over previous
import jax
import jax.numpy as jnp
from jax import lax
from jax.experimental import pallas as pl
from jax.experimental.pallas import tpu as pltpu
from jax.experimental.pallas import tpu_sc as plsc

NC = 2
NS = 16
NW = NC * NS
LANES = 16
CHUNK = 80
JB = 25


def _make_sc_segsum(n, e, d, with_deg):
    chunks_per_tile = e // (NW * CHUNK)
    outer = chunks_per_tile // JB
    rows_per_tile = n // NS

    mesh = plsc.VectorSubcoreMesh(core_axis_name="c", subcore_axis_name="s")

    out_type = [jax.ShapeDtypeStruct((NC, NS, rows_per_tile, d), jnp.float32)]
    scratch = [
        pltpu.VMEM((JB, CHUNK), jnp.int32),
        pltpu.VMEM((JB, CHUNK), jnp.int32),
        pltpu.VMEM((2, CHUNK, d), jnp.float32),
        pltpu.VMEM_SHARED((n, d), jnp.float32),
        pltpu.SemaphoreType.DMA((2,)),
    ]
    if with_deg:
        out_type.append(jax.ShapeDtypeStruct((NW, n), jnp.float32))
        scratch.append(pltpu.VMEM((n,), jnp.float32))

    def body(x_hbm, rows_hbm, cols_hbm, zrows_hbm, zn_hbm, agg_out, deg_out,
             rowbuf, colbuf, databuf, aggs, sem, degbuf):
        cid = lax.axis_index("c")
        sid = lax.axis_index("s")
        wid = cid * NS + sid
        r0 = sid * rows_per_tile
        pltpu.sync_copy(zrows_hbm, aggs.at[pl.ds(r0, rows_per_tile)])
        if with_deg:
            pltpu.sync_copy(zn_hbm, degbuf)
        plsc.subcore_barrier()

        ones = jnp.full((LANES,), 1.0, jnp.float32)

        def outer_body(ob, carry):
            pltpu.sync_copy(rows_hbm.at[wid, ob], rowbuf)
            pltpu.sync_copy(cols_hbm.at[wid, ob], colbuf)

            pltpu.async_copy(x_hbm.at[colbuf.at[0]], databuf.at[0],
                             sem.at[0])

            def inner(j, carry2):
                p = lax.rem(j, 2)
                q = lax.rem(j + 1, 2)

                @pl.when(j + 1 < JB)
                def _():
                    pltpu.async_copy(x_hbm.at[colbuf.at[j + 1]],
                                     databuf.at[q], sem.at[q])

                if with_deg:
                    for i in range(CHUNK // LANES):
                        rv = rowbuf[j, pl.ds(i * LANES, LANES)]
                        plsc.addupdate_scatter(degbuf, [rv], ones)

                pltpu.make_async_copy(x_hbm.at[colbuf.at[j]],
                                      databuf.at[p], sem.at[p]).wait()
                pltpu.sync_copy(databuf.at[p], aggs.at[rowbuf.at[j]],
                                add=True)
                return carry2
            lax.fori_loop(0, JB, inner, 0)
            return carry
        lax.fori_loop(0, outer, outer_body, 0)

        if with_deg:
            pltpu.sync_copy(degbuf, deg_out.at[wid])
        plsc.subcore_barrier()
        pltpu.sync_copy(aggs.at[pl.ds(r0, rows_per_tile)],
                        agg_out.at[cid, sid])

    if with_deg:
        def full_body(x_hbm, rows_hbm, cols_hbm, zrows_hbm, zn_hbm,
                      agg_out, deg_out, rowbuf, colbuf, databuf, aggs, sem,
                      degbuf):
            body(x_hbm, rows_hbm, cols_hbm, zrows_hbm, zn_hbm, agg_out,
                 deg_out, rowbuf, colbuf, databuf, aggs, sem, degbuf)
    else:
        def full_body(x_hbm, rows_hbm, cols_hbm, zrows_hbm, agg_out,
                      rowbuf, colbuf, databuf, aggs, sem):
            body(x_hbm, rows_hbm, cols_hbm, zrows_hbm, None, agg_out,
                 None, rowbuf, colbuf, databuf, aggs, sem, None)

    return pl.kernel(full_body, out_type=out_type, mesh=mesh,
                     scratch_types=scratch,
                     compiler_params=pltpu.CompilerParams(
                         needs_layout_passes=False,
                         use_tc_tiling_on_sc=False))


_BN = 1000


def _tc_mid(aggp, degp_t, x, b1, w1, w2):
    n, d = x.shape
    dout = w2.shape[1]

    def body(aggp_ref, degp_ref, x_ref, b_ref, w1_ref, w2_ref, o_ref,
             dinv_ref):
        agg = aggp_ref[0] + aggp_ref[1]
        deginv = 1.0 / jnp.maximum(jnp.sum(degp_ref[...], axis=1), 1.0)
        z = agg * deginv[:, None] + x_ref[...]
        h = jnp.dot(z, w1_ref[...], preferred_element_type=jnp.float32)
        h = jnp.maximum(h + b_ref[...], 0.0)
        o_ref[...] = jnp.dot(h, w2_ref[...],
                             preferred_element_type=jnp.float32)
        dinv_ref[...] = deginv[:, None]

    return pl.pallas_call(
        body,
        grid=(n // _BN,),
        in_specs=[
            pl.BlockSpec((NC, _BN, d), lambda i: (0, i, 0)),
            pl.BlockSpec((_BN, NW), lambda i: (i, 0)),
            pl.BlockSpec((_BN, d), lambda i: (i, 0)),
            pl.BlockSpec((1, d), lambda i: (0, 0)),
            pl.BlockSpec((d, d), lambda i: (0, 0)),
            pl.BlockSpec((d, dout), lambda i: (0, 0)),
        ],
        out_specs=[
            pl.BlockSpec((_BN, dout), lambda i: (i, 0)),
            pl.BlockSpec((_BN, 1), lambda i: (i, 0)),
        ],
        out_shape=[
            jax.ShapeDtypeStruct((n, dout), jnp.float32),
            jax.ShapeDtypeStruct((n, 1), jnp.float32),
        ],
    )(aggp, degp_t, x, b1, w1, w2)


def _tc_final(aggp, deginv, y2, b2):
    n, d = y2.shape

    def body(aggp_ref, dinv_ref, y_ref, b_ref, o_ref):
        agg = aggp_ref[0] + aggp_ref[1]
        o_ref[...] = agg * dinv_ref[...] + y_ref[...] + b_ref[...]

    return pl.pallas_call(
        body,
        grid=(n // _BN,),
        in_specs=[
            pl.BlockSpec((NC, _BN, d), lambda i: (0, i, 0)),
            pl.BlockSpec((_BN, 1), lambda i: (i, 0)),
            pl.BlockSpec((_BN, d), lambda i: (i, 0)),
            pl.BlockSpec((1, d), lambda i: (0, 0)),
        ],
        out_specs=pl.BlockSpec((_BN, d), lambda i: (i, 0)),
        out_shape=jax.ShapeDtypeStruct((n, d), jnp.float32),
    )(aggp, deginv, y2, b2)


def kernel(x, edge_index, W1, b1, W2, b2):
    n, d = x.shape
    e = edge_index.shape[1]
    d2 = W2.shape[1]
    cpt = e // (NW * CHUNK)
    rows = edge_index[0].reshape(NW, cpt // JB, JB, CHUNK)
    cols = edge_index[1].reshape(NW, cpt // JB, JB, CHUNK)
    zrows = jnp.zeros((n // NS, d), jnp.float32)
    zrows2 = jnp.zeros((n // NS, d2), jnp.float32)
    zn = jnp.zeros((n,), jnp.float32)

    agg1, degp = _make_sc_segsum(n, e, d, True)(x, rows, cols, zrows, zn)
    y2, deginv = _tc_mid(agg1.reshape(NC, n, d), degp.T, x,
                         b1.reshape(1, -1), W1, W2)
    (agg2,) = _make_sc_segsum(n, e, d2, False)(y2, rows, cols, zrows2)
    out = _tc_final(agg2.reshape(NC, n, d2), deginv, y2, b2.reshape(1, -1))
    return out
